# Initial kernel scaffold; baseline (speedup 1.0000x reference)
import jax
import jax.numpy as jnp
from jax.experimental import pallas as pl


def kernel(x, edge_index, W_env_self, W_env_nbr, b_env, W_act_self, W_act_nbr, b_act, W_cls, b_cls):
    raise NotImplementedError("write your pallas kernel here")


# speedup vs baseline: 3.3363x; 3.3363x over previous
import functools

import jax
import jax.numpy as jnp
from jax import lax
from jax.experimental import pallas as pl
from jax.experimental.pallas import tpu as pltpu
from jax.experimental.pallas import tpu_sc as plsc

_NC = 2
_NS = 16
_NW = _NC * _NS
_CHUNK = 128


def _round_up(a, b):
    return (a + b - 1) // b * b


@functools.partial(jax.jit, static_argnums=(3, 4))
def _sc_segment_sum(x, src, dst, n_pad, rpt):
    _, D = x.shape
    ept = src.shape[0] // _NW
    cpt = ept // _CHUNK
    zrows_n = rpt // 8
    zvec_n = _round_up(rpt, 16)

    mesh = plsc.VectorSubcoreMesh(core_axis_name="c", subcore_axis_name="s")

    @functools.partial(
        pl.kernel,
        mesh=mesh,
        out_type=(
            jax.ShapeDtypeStruct((_NC * n_pad, D), jnp.float32),
            jax.ShapeDtypeStruct((_NC * n_pad,), jnp.float32),
        ),
        scratch_types=[
            pltpu.VMEM((_CHUNK,), jnp.int32),
            pltpu.VMEM((_CHUNK,), jnp.int32),
            pltpu.VMEM((_CHUNK, D), jnp.float32),
            pltpu.VMEM((_CHUNK,), jnp.float32),
            pltpu.VMEM((zrows_n, D), jnp.float32),
            pltpu.VMEM((zvec_n,), jnp.float32),
            pltpu.VMEM_SHARED((n_pad, D), jnp.float32),
            pltpu.VMEM_SHARED((n_pad,), jnp.float32),
            pltpu.SemaphoreType.DMA,
        ],
    )
    def seg_sum(x_hbm, src_hbm, dst_hbm, s_out, cnt_out,
                sidx, didx, rows, ones_v, zrows, zvec, s_acc, c_acc, sem):
        cid = lax.axis_index("c")
        sid = lax.axis_index("s")
        wid = sid * _NC + cid

        zero16 = jnp.zeros((16,), jnp.float32)
        one16 = jnp.ones((16,), jnp.float32)
        for j in range(_CHUNK // 16):
            ones_v[pl.ds(j * 16, 16)] = one16

        def zv_body(i, _):
            zvec[pl.ds(i * 16, 16)] = zero16
            return 0
        lax.fori_loop(0, zvec_n // 16, zv_body, 0)

        def zr_body(i, _):
            for j in range(D // 16):
                zrows[i, pl.ds(j * 16, 16)] = zero16
            return 0
        lax.fori_loop(0, zrows_n, zr_body, 0)

        row0 = sid * rpt
        for k in range(rpt // zrows_n):
            pltpu.sync_copy(zrows, s_acc.at[pl.ds(row0 + k * zrows_n, zrows_n), :])
        pltpu.sync_copy(zvec.at[pl.ds(0, rpt)], c_acc.at[pl.ds(row0, rpt)])
        plsc.subcore_barrier()

        ebase = wid * ept

        def chunk_body(ci, _):
            off = ebase + ci * _CHUNK
            pltpu.sync_copy(src_hbm.at[pl.ds(off, _CHUNK)], sidx)
            pltpu.sync_copy(dst_hbm.at[pl.ds(off, _CHUNK)], didx)
            pltpu.async_copy(x_hbm.at[sidx], rows, sem).wait()
            pltpu.sync_copy(rows, s_acc.at[didx], add=True)
            pltpu.sync_copy(ones_v, c_acc.at[didx], add=True)
            return 0
        lax.fori_loop(0, cpt, chunk_body, 0)
        plsc.subcore_barrier()

        obase = cid * n_pad + row0
        pltpu.sync_copy(s_acc.at[pl.ds(row0, rpt), :],
                        s_out.at[pl.ds(obase, rpt), :])
        pltpu.sync_copy(c_acc.at[pl.ds(row0, rpt)], zvec.at[pl.ds(0, rpt)])
        pltpu.sync_copy(zvec.at[pl.ds(0, rpt)], cnt_out.at[pl.ds(obase, rpt)])

    return seg_sum(x, src, dst)


def _tc_dense(x, s2, cnt2, w_nbr, w_self, b_env, w_cls_p, b_cls_p, C):
    N, D = x.shape
    R = 2000
    grid = (N // R,)

    def body(x_ref, s_ref, c_ref, wn_ref, ws_ref, be_ref, wc_ref, bc_ref, o_ref):
        s = s_ref[0] + s_ref[1]
        cnt = c_ref[0] + c_ref[1]
        agg = s / jnp.maximum(cnt, 1.0)
        agg = jnp.dot(agg, wn_ref[...], preferred_element_type=jnp.float32)
        h = jnp.dot(x_ref[...], ws_ref[...], preferred_element_type=jnp.float32)
        x1 = jax.nn.gelu(h + agg + be_ref[...])
        logits = jnp.dot(x1, wc_ref[...], preferred_element_type=jnp.float32)
        g = jax.nn.gelu(logits + bc_ref[...])
        col = lax.broadcasted_iota(jnp.int32, g.shape, 1)
        gm = jnp.where(col < C, g, -1e30)
        m = jnp.max(gm, axis=1, keepdims=True)
        e = jnp.exp(gm - m)
        o_ref[...] = e / jnp.sum(e, axis=1, keepdims=True)

    Dp = w_cls_p.shape[1]
    return pl.pallas_call(
        body,
        grid=grid,
        in_specs=[
            pl.BlockSpec((R, D), lambda i: (i, 0)),
            pl.BlockSpec((2, R, D), lambda i: (0, i, 0)),
            pl.BlockSpec((2, R, 1), lambda i: (0, i, 0)),
            pl.BlockSpec((D, D), lambda i: (0, 0)),
            pl.BlockSpec((D, D), lambda i: (0, 0)),
            pl.BlockSpec((1, D), lambda i: (0, 0)),
            pl.BlockSpec((D, Dp), lambda i: (0, 0)),
            pl.BlockSpec((1, Dp), lambda i: (0, 0)),
        ],
        out_specs=pl.BlockSpec((R, Dp), lambda i: (i, 0)),
        out_shape=jax.ShapeDtypeStruct((N, Dp), jnp.float32),
    )(x, s2, cnt2, w_nbr, w_self, b_env, w_cls_p, b_cls_p)


def kernel(x, edge_index, W_env_self, W_env_nbr, b_env,
           W_act_self, W_act_nbr, b_act, W_cls, b_cls):
    N, D = x.shape
    C = W_cls.shape[1]
    E = edge_index.shape[1]

    cpt = _round_up(-(-E // _NW) , 2 * _CHUNK) // _CHUNK
    e_pad = _NW * cpt * _CHUNK
    rpt = _round_up(-(-(N + 1) // _NS), 8)
    n_pad = rpt * _NS

    pad = e_pad - E
    src = jnp.concatenate([edge_index[0], jnp.zeros((pad,), jnp.int32)])
    dst = jnp.concatenate([edge_index[1], jnp.full((pad,), N, jnp.int32)])

    s_flat, cnt_flat = _sc_segment_sum(x, src, dst, n_pad, rpt)
    s2 = s_flat.reshape(_NC, n_pad, D)
    cnt2 = cnt_flat.reshape(_NC, n_pad, 1)

    Dp = _round_up(C, 128)
    w_cls_p = jnp.pad(W_cls, ((0, 0), (0, Dp - C)))
    b_cls_p = jnp.pad(b_cls, (0, Dp - C)).reshape(1, Dp)

    out = _tc_dense(x, s2, cnt2, W_env_nbr, W_env_self,
                    b_env.reshape(1, D), w_cls_p, b_cls_p, C)
    return out[:, :C]
